# Initial kernel scaffold; baseline (speedup 1.0000x reference)
import jax
import jax.numpy as jnp
from jax.experimental import pallas as pl


def kernel(v, edges, batch, W1a, b1a, W2a, b2a, eps1, W1b, b1b, W2b, b2b, eps2):
    raise NotImplementedError("write your pallas kernel here")


# speedup vs baseline: 4.8172x; 4.8172x over previous
import functools

import jax
import jax.numpy as jnp
from jax import lax
from jax.experimental import pallas as pl
from jax.experimental.pallas import tpu as pltpu
from jax.experimental.pallas import tpu_sc as plsc

NC = 2
NS = 16
CH = 80


def _sc_segsum(x, src, dst, zeros, n, d, e):
    nw = NC * NS
    epw = e // nw
    nchunk = epw // CH
    rows_pt = (n // (8 * NS)) * 8
    tail = n - rows_pt * NS

    mesh = plsc.VectorSubcoreMesh(core_axis_name="c", subcore_axis_name="s")

    @functools.partial(
        pl.kernel,
        out_type=jax.ShapeDtypeStruct((NC, n, d), jnp.float32),
        mesh=mesh,
        scratch_types=[
            pltpu.VMEM_SHARED((n, d), jnp.float32),
            pltpu.VMEM((CH,), jnp.int32),
            pltpu.VMEM((CH,), jnp.int32),
            pltpu.VMEM((CH, d), jnp.float32),
            pltpu.SemaphoreType.DMA,
        ],
    )
    def k(x_hbm, src_hbm, dst_hbm, zeros_hbm, out_hbm, agg_sh, src_v, dst_v,
          rows_v, sem):
        c = lax.axis_index("c")
        s = lax.axis_index("s")
        wid = c * NS + s
        r0 = pl.multiple_of(s * rows_pt, 8)
        pltpu.sync_copy(zeros_hbm.at[pl.ds(r0, rows_pt)],
                        agg_sh.at[pl.ds(r0, rows_pt)])
        if tail:
            @pl.when(s == NS - 1)
            def _():
                t0 = NS * rows_pt
                pltpu.sync_copy(zeros_hbm.at[pl.ds(t0, tail)],
                                agg_sh.at[pl.ds(t0, tail)])
        plsc.subcore_barrier()

        base0 = wid * epw

        def step(j, carry):
            b = pl.multiple_of(base0 + j * CH, 8)
            pltpu.sync_copy(src_hbm.at[pl.ds(b, CH)], src_v)
            pltpu.sync_copy(dst_hbm.at[pl.ds(b, CH)], dst_v)
            pltpu.async_copy(x_hbm.at[src_v], rows_v, sem).wait()
            pltpu.sync_copy(rows_v, agg_sh.at[dst_v], add=True)
            return carry

        lax.fori_loop(0, nchunk, step, 0)
        plsc.subcore_barrier()
        pltpu.sync_copy(agg_sh.at[pl.ds(r0, rows_pt)],
                        out_hbm.at[c, pl.ds(r0, rows_pt)])
        if tail:
            @pl.when(s == NS - 1)
            def _():
                t0 = NS * rows_pt
                pltpu.sync_copy(agg_sh.at[pl.ds(t0, tail)],
                                out_hbm.at[c, pl.ds(t0, tail)])

    return k(x, src, dst, zeros)


def _tc_gin_layer(x, agg, w1, b1, w2, b2, eps, *, final_relu, blk, n, d):
    nblk = n // blk

    def body(eps_ref, x_ref, a0_ref, a1_ref, w1_ref, b1_ref, w2_ref, b2_ref,
             o_ref):
        z = (1.0 + eps_ref[0]) * x_ref[...] + a0_ref[0] + a1_ref[0]
        t = jnp.dot(z, w1_ref[...], preferred_element_type=jnp.float32)
        t = jnp.maximum(t + b1_ref[...], 0.0)
        h = jnp.dot(t, w2_ref[...], preferred_element_type=jnp.float32)
        h = h + b2_ref[...]
        if final_relu:
            h = jnp.maximum(h, 0.0)
        o_ref[...] = h

    wspec = pl.BlockSpec((d, d), lambda i: (0, 0))
    bspec = pl.BlockSpec((1, d), lambda i: (0, 0))
    return pl.pallas_call(
        functools.partial(body),
        grid=(nblk,),
        in_specs=[
            pl.BlockSpec(memory_space=pltpu.SMEM),
            pl.BlockSpec((blk, d), lambda i: (i, 0)),
            pl.BlockSpec((1, blk, d), lambda i: (0, i, 0)),
            pl.BlockSpec((1, blk, d), lambda i: (1, i, 0)),
            wspec, bspec, wspec, bspec,
        ],
        out_specs=pl.BlockSpec((blk, d), lambda i: (i, 0)),
        out_shape=jax.ShapeDtypeStruct((n, d), jnp.float32),
        compiler_params=pltpu.CompilerParams(
            dimension_semantics=("arbitrary",)),
    )(eps.reshape(1), x, agg, agg, w1, b1.reshape(1, d), w2,
      b2.reshape(1, d))


def _tc_gin_pool(x, agg, batch3, w1, b1, w2, b2, eps, *, blk, n, d, g):
    nblk = n // blk

    def body(eps_ref, x_ref, a0_ref, a1_ref, w1_ref, b1_ref, w2_ref, b2_ref,
             batch_ref, pooled_ref, sig_ref, sums_scr, counts_scr):
        i = pl.program_id(0)
        z = (1.0 + eps_ref[0]) * x_ref[...] + a0_ref[0] + a1_ref[0]
        t = jnp.dot(z, w1_ref[...], preferred_element_type=jnp.float32)
        t = jnp.maximum(t + b1_ref[...], 0.0)
        h = jnp.dot(t, w2_ref[...], preferred_element_type=jnp.float32)
        h = h + b2_ref[...]
        seg = batch_ref[0, 0, :].reshape(blk, 1)
        ids = lax.broadcasted_iota(jnp.int32, (1, g), 1)
        onehot = (seg == ids).astype(jnp.float32)
        ps = lax.dot_general(onehot, h, (((0,), (0,)), ((), ())),
                             preferred_element_type=jnp.float32)
        pc = jnp.sum(onehot, axis=0).reshape(g, 1)

        @pl.when(i == 0)
        def _():
            sums_scr[...] = ps
            counts_scr[...] = pc

        @pl.when(i > 0)
        def _():
            sums_scr[...] += ps
            counts_scr[...] += pc

        @pl.when(i == nblk - 1)
        def _():
            pooled = sums_scr[...] / jnp.maximum(counts_scr[...], 1.0)
            pooled_ref[...] = pooled
            sig_ref[...] = jax.nn.sigmoid(pooled)

    wspec = pl.BlockSpec((d, d), lambda i: (0, 0))
    bspec = pl.BlockSpec((1, d), lambda i: (0, 0))
    gspec = pl.BlockSpec((g, d), lambda i: (0, 0))
    return pl.pallas_call(
        body,
        grid=(nblk,),
        in_specs=[
            pl.BlockSpec(memory_space=pltpu.SMEM),
            pl.BlockSpec((blk, d), lambda i: (i, 0)),
            pl.BlockSpec((1, blk, d), lambda i: (0, i, 0)),
            pl.BlockSpec((1, blk, d), lambda i: (1, i, 0)),
            wspec, bspec, wspec, bspec,
            pl.BlockSpec((1, 1, blk), lambda i: (i, 0, 0)),
        ],
        out_specs=[gspec, gspec],
        out_shape=[jax.ShapeDtypeStruct((g, d), jnp.float32),
                   jax.ShapeDtypeStruct((g, d), jnp.float32)],
        scratch_shapes=[pltpu.VMEM((g, d), jnp.float32),
                        pltpu.VMEM((g, 1), jnp.float32)],
        compiler_params=pltpu.CompilerParams(
            dimension_semantics=("arbitrary",)),
    )(eps.reshape(1), x, agg, agg, w1, b1.reshape(1, d), w2,
      b2.reshape(1, d), batch3)


def kernel(v, edges, batch, W1a, b1a, W2a, b2a, eps1, W1b, b1b, W2b, b2b,
           eps2):
    n, d = v.shape
    e = edges.shape[1]
    g = 256
    blk = 1000

    src = edges[0]
    dst = edges[1]
    zeros = jnp.zeros((n, d), jnp.float32)
    batch3 = batch.reshape(n // blk, 1, blk)

    agg1 = _sc_segsum(v, src, dst, zeros, n, d, e)
    x1 = _tc_gin_layer(v, agg1, W1a, b1a, W2a, b2a, eps1,
                       final_relu=True, blk=blk, n=n, d=d)
    agg2 = _sc_segsum(x1, src, dst, zeros, n, d, e)
    pooled, sig = _tc_gin_pool(x1, agg2, batch3, W1b, b1b, W2b, b2b, eps2,
                               blk=blk, n=n, d=d, g=g)
    return (pooled, sig)


# speedup vs baseline: 11.0035x; 2.2842x over previous
import functools

import jax
import jax.numpy as jnp
from jax import lax
from jax.experimental import pallas as pl
from jax.experimental.pallas import tpu as pltpu
from jax.experimental.pallas import tpu_sc as plsc

NC = 2
NS = 16
CH = 80


def _sc_segsum(x, src3, dst, zeros, n, d, e):
    nw = NC * NS
    epw = e // nw
    nchunk = epw // CH
    npair = (nchunk - 1) // 2
    nrem = nchunk - 2 * npair
    rows_pt = (n // (8 * NS)) * 8
    tail = n - rows_pt * NS

    mesh = plsc.VectorSubcoreMesh(core_axis_name="c", subcore_axis_name="s")

    @functools.partial(
        pl.kernel,
        out_type=jax.ShapeDtypeStruct((NC, n, d), jnp.float32),
        mesh=mesh,
        scratch_types=[
            pltpu.VMEM_SHARED((n, d), jnp.float32),
            pltpu.VMEM((nchunk, CH), jnp.int32),
            pltpu.VMEM((CH,), jnp.int32),
            pltpu.VMEM((CH,), jnp.int32),
            pltpu.VMEM((CH, d), jnp.float32),
            pltpu.VMEM((CH, d), jnp.float32),
            pltpu.SemaphoreType.DMA,
            pltpu.SemaphoreType.DMA,
            pltpu.SemaphoreType.DMA,
            pltpu.SemaphoreType.DMA,
        ],
    )
    def k(x_hbm, src_hbm, dst_hbm, zeros_hbm, out_hbm, agg_sh, src_v,
          dstc_a, dstc_b, rows_a, rows_b, sem_a, sem_b, sem_da, sem_db):
        c = lax.axis_index("c")
        s = lax.axis_index("s")
        wid = c * NS + s
        r0 = pl.multiple_of(s * rows_pt, 8)
        base0 = wid * epw
        idx_a = pltpu.async_copy(src_hbm.at[wid], src_v, sem_a)
        pltpu.sync_copy(zeros_hbm.at[pl.ds(r0, rows_pt)],
                        agg_sh.at[pl.ds(r0, rows_pt)])
        if tail:
            @pl.when(s == NS - 1)
            def _():
                t0 = NS * rows_pt
                pltpu.sync_copy(zeros_hbm.at[pl.ds(t0, tail)],
                                agg_sh.at[pl.ds(t0, tail)])
        idx_a.wait()
        plsc.subcore_barrier()

        last = nchunk - 1

        def gather(chunk, rows, sem):
            pltpu.async_copy(x_hbm.at[src_v.at[chunk]], rows, sem)

        def gwait(rows, sem):
            pltpu.make_async_copy(x_hbm.at[src_v.at[0]], rows, sem).wait()

        def dload(chunk, dstc, sem):
            b = pl.multiple_of(base0 + chunk * CH, 8)
            pltpu.async_copy(dst_hbm.at[pl.ds(b, CH)], dstc, sem)

        def dwait(dstc, sem):
            pltpu.make_async_copy(dst_hbm.at[pl.ds(0, CH)], dstc, sem).wait()

        def scatter(dstc, rows):
            pltpu.sync_copy(rows, agg_sh.at[dstc], add=True)

        gather(0, rows_a, sem_a)
        gather(1, rows_b, sem_b)
        dload(0, dstc_a, sem_da)
        dload(1, dstc_b, sem_db)

        def step(j, carry):
            ca = 2 * j
            gwait(rows_a, sem_a)
            dwait(dstc_a, sem_da)
            scatter(dstc_a, rows_a)
            gather(jnp.minimum(ca + 2, last), rows_a, sem_a)
            dload(jnp.minimum(ca + 2, last), dstc_a, sem_da)
            gwait(rows_b, sem_b)
            dwait(dstc_b, sem_db)
            scatter(dstc_b, rows_b)
            gather(jnp.minimum(ca + 3, last), rows_b, sem_b)
            dload(jnp.minimum(ca + 3, last), dstc_b, sem_db)
            return carry

        lax.fori_loop(0, npair, step, 0)
        gwait(rows_a, sem_a)
        dwait(dstc_a, sem_da)
        if nrem == 1:
            scatter(dstc_a, rows_a)
        gwait(rows_b, sem_b)
        dwait(dstc_b, sem_db)
        if nrem == 2:
            scatter(dstc_a, rows_a)
            scatter(dstc_b, rows_b)
        plsc.subcore_barrier()
        pltpu.sync_copy(agg_sh.at[pl.ds(r0, rows_pt)],
                        out_hbm.at[c, pl.ds(r0, rows_pt)])
        if tail:
            @pl.when(s == NS - 1)
            def _():
                t0 = NS * rows_pt
                pltpu.sync_copy(agg_sh.at[pl.ds(t0, tail)],
                                out_hbm.at[c, pl.ds(t0, tail)])

    return k(x, src3, dst, zeros)


def _tc_gin_layer(x, agg, w1, b1, w2, b2, eps, *, final_relu, blk, n, d):
    nblk = n // blk

    def body(eps_ref, x_ref, a0_ref, a1_ref, w1_ref, b1_ref, w2_ref, b2_ref,
             o_ref):
        z = (1.0 + eps_ref[0]) * x_ref[...] + a0_ref[0] + a1_ref[0]
        t = jnp.dot(z, w1_ref[...], preferred_element_type=jnp.float32)
        t = jnp.maximum(t + b1_ref[...], 0.0)
        h = jnp.dot(t, w2_ref[...], preferred_element_type=jnp.float32)
        h = h + b2_ref[...]
        if final_relu:
            h = jnp.maximum(h, 0.0)
        o_ref[...] = h

    wspec = pl.BlockSpec((d, d), lambda i: (0, 0))
    bspec = pl.BlockSpec((1, d), lambda i: (0, 0))
    return pl.pallas_call(
        functools.partial(body),
        grid=(nblk,),
        in_specs=[
            pl.BlockSpec(memory_space=pltpu.SMEM),
            pl.BlockSpec((blk, d), lambda i: (i, 0)),
            pl.BlockSpec((1, blk, d), lambda i: (0, i, 0)),
            pl.BlockSpec((1, blk, d), lambda i: (1, i, 0)),
            wspec, bspec, wspec, bspec,
        ],
        out_specs=pl.BlockSpec((blk, d), lambda i: (i, 0)),
        out_shape=jax.ShapeDtypeStruct((n, d), jnp.float32),
        compiler_params=pltpu.CompilerParams(
            dimension_semantics=("arbitrary",)),
    )(eps.reshape(1), x, agg, agg, w1, b1.reshape(1, d), w2,
      b2.reshape(1, d))


def _tc_gin_pool(x, agg, batch3, w1, b1, w2, b2, eps, *, blk, n, d, g):
    nblk = n // blk

    def body(eps_ref, x_ref, a0_ref, a1_ref, w1_ref, b1_ref, w2_ref, b2_ref,
             batch_ref, pooled_ref, sig_ref, sums_scr, counts_scr):
        i = pl.program_id(0)
        z = (1.0 + eps_ref[0]) * x_ref[...] + a0_ref[0] + a1_ref[0]
        t = jnp.dot(z, w1_ref[...], preferred_element_type=jnp.float32)
        t = jnp.maximum(t + b1_ref[...], 0.0)
        h = jnp.dot(t, w2_ref[...], preferred_element_type=jnp.float32)
        h = h + b2_ref[...]
        seg = batch_ref[0, 0, :].reshape(blk, 1)
        ids = lax.broadcasted_iota(jnp.int32, (1, g), 1)
        onehot = (seg == ids).astype(jnp.float32)
        ps = lax.dot_general(onehot, h, (((0,), (0,)), ((), ())),
                             preferred_element_type=jnp.float32)
        pc = jnp.sum(onehot, axis=0).reshape(g, 1)

        @pl.when(i == 0)
        def _():
            sums_scr[...] = ps
            counts_scr[...] = pc

        @pl.when(i > 0)
        def _():
            sums_scr[...] += ps
            counts_scr[...] += pc

        @pl.when(i == nblk - 1)
        def _():
            pooled = sums_scr[...] / jnp.maximum(counts_scr[...], 1.0)
            pooled_ref[...] = pooled
            sig_ref[...] = jax.nn.sigmoid(pooled)

    wspec = pl.BlockSpec((d, d), lambda i: (0, 0))
    bspec = pl.BlockSpec((1, d), lambda i: (0, 0))
    gspec = pl.BlockSpec((g, d), lambda i: (0, 0))
    return pl.pallas_call(
        body,
        grid=(nblk,),
        in_specs=[
            pl.BlockSpec(memory_space=pltpu.SMEM),
            pl.BlockSpec((blk, d), lambda i: (i, 0)),
            pl.BlockSpec((1, blk, d), lambda i: (0, i, 0)),
            pl.BlockSpec((1, blk, d), lambda i: (1, i, 0)),
            wspec, bspec, wspec, bspec,
            pl.BlockSpec((1, 1, blk), lambda i: (i, 0, 0)),
        ],
        out_specs=[gspec, gspec],
        out_shape=[jax.ShapeDtypeStruct((g, d), jnp.float32),
                   jax.ShapeDtypeStruct((g, d), jnp.float32)],
        scratch_shapes=[pltpu.VMEM((g, d), jnp.float32),
                        pltpu.VMEM((g, 1), jnp.float32)],
        compiler_params=pltpu.CompilerParams(
            dimension_semantics=("arbitrary",)),
    )(eps.reshape(1), x, agg, agg, w1, b1.reshape(1, d), w2,
      b2.reshape(1, d), batch3)


def kernel(v, edges, batch, W1a, b1a, W2a, b2a, eps1, W1b, b1b, W2b, b2b,
           eps2):
    n, d = v.shape
    e = edges.shape[1]
    g = 256
    blk = 1000

    nw = NC * NS
    nchunk = e // (nw * CH)
    src3 = edges[0].reshape(nw, nchunk, CH)
    dst = edges[1]
    zeros = jnp.zeros((n, d), jnp.float32)
    batch3 = batch.reshape(n // blk, 1, blk)

    agg1 = _sc_segsum(v, src3, dst, zeros, n, d, e)
    x1 = _tc_gin_layer(v, agg1, W1a, b1a, W2a, b2a, eps1,
                       final_relu=True, blk=blk, n=n, d=d)
    agg2 = _sc_segsum(x1, src3, dst, zeros, n, d, e)
    pooled, sig = _tc_gin_pool(x1, agg2, batch3, W1b, b1b, W2b, b2b, eps2,
                               blk=blk, n=n, d=d, g=g)
    return (pooled, sig)
